# baseline (speedup 1.0000x reference)
import functools

import jax
import jax.numpy as jnp
from jax import lax
from jax.experimental import pallas as pl
from jax.experimental.pallas import tpu as pltpu
from jax.experimental.pallas import tpu_sc as plsc

_R = 0.86
_L = 16
_NW = 32
_CH = 1024
_NB = 2048
_HPAD = 2056
_NCAND = 1024
_LN2 = 0.6931471805599453


def _ilog_poly(s):
    bits = plsc.bitcast(s, jnp.int32)
    ex = lax.shift_right_logical(bits, 23) - 127
    mb = lax.bitwise_or(lax.bitwise_and(bits, 0x7FFFFF), 0x3F800000)
    m = plsc.bitcast(mb, jnp.float32)
    z = (m - 1.0) / (m + 1.0)
    z2 = z * z
    p = 1.0 / 9.0 + z2 * 0.0
    p = 1.0 / 7.0 + z2 * p
    p = 1.0 / 5.0 + z2 * p
    p = 1.0 / 3.0 + z2 * p
    p = 1.0 + z2 * p
    return ex.astype(jnp.float32) * _LN2 + 2.0 * z * p


def _search(hist, a0, target, nbins, lane):
    lo = jnp.zeros((_L,), jnp.int32)
    hi = jnp.full((_L,), nbins - 1, jnp.int32)
    steps = nbins.bit_length() - 1

    def step(_, carry):
        lo, hi = carry
        mid = lax.shift_right_logical(lo + hi + 1, 1)
        sv = plsc.load_gather(hist, [mid, lane])
        c = (a0 + sv) >= target
        return jnp.where(c, mid, lo), jnp.where(c, hi, mid - 1)

    lo, hi = lax.fori_loop(0, steps, step, (lo, hi))
    anext = a0 + plsc.load_gather(hist, [lo + 1, lane])
    return lo, anext


def _zero_hist(hist, nrows):
    def z(j, _):
        hist[j] = jnp.zeros((_L,), jnp.float32)
        return 0

    lax.fori_loop(0, nrows, z, 0, unroll=8)


def _suffix_sum(hist, nbins):
    def sfx(j, acc):
        jr = nbins - 1 - j
        acc = acc + hist[jr]
        hist[jr] = acc
        return acc

    return lax.fori_loop(0, nbins, sfx, jnp.zeros((_L,), jnp.float32),
                         unroll=4)


def _sc_body(x_hbm, o_hbm, buf, obuf, hist, cand_e, cand_k):
    wid = lax.axis_index("s") * 2 + lax.axis_index("c")
    lane = lax.iota(jnp.int32, _L)
    B, V, C = 64, 8192, 32
    nchunks = V // _CH

    def run_unit(t, _):
        unit = t * _NW + wid
        b = lax.shift_right_logical(unit, 1)
        h = lax.bitwise_and(unit, 1)
        col0 = h * _L

        _zero_hist(hist, _HPAD)

        def scan1_chunk(ci, _):
            pltpu.sync_copy(
                x_hbm.at[b, pl.ds(ci * _CH, _CH), pl.ds(col0, _L)], buf)

            def row(i, _):
                e = jnp.exp(buf[i])
                k = plsc.bitcast(e, jnp.int32)
                b1 = lax.shift_right_logical(k, 20)
                plsc.addupdate_scatter(hist, [b1, lane], e)
                return 0

            return lax.fori_loop(0, _CH, row, 0, unroll=8)

        lax.fori_loop(0, nchunks, scan1_chunk, 0)

        s = _suffix_sum(hist, _NB)
        target = _R * s
        b1s, a1 = _search(hist, jnp.zeros((_L,), jnp.float32), target, _NB, lane)

        def scan2_chunk(ci, cnt):
            pltpu.sync_copy(
                x_hbm.at[b, pl.ds(ci * _CH, _CH), pl.ds(col0, _L)], buf)

            def row(i, cnt):
                e = jnp.exp(buf[i])
                k = plsc.bitcast(e, jnp.int32)
                m = (lax.shift_right_logical(k, 20) == b1s) & (cnt < _NCAND)
                plsc.store_scatter(cand_e, [cnt, lane], e, mask=m)
                plsc.store_scatter(cand_k, [cnt, lane], k, mask=m)
                return cnt + jnp.where(m, 1, 0)

            return lax.fori_loop(0, _CH, row, cnt, unroll=8)

        cnt = lax.fori_loop(0, nchunks, scan2_chunk, jnp.zeros((_L,), jnp.int32))

        nmax = jnp.max(cnt)

        _zero_hist(hist, _HPAD)

        def r2(j, _):
            valid = (lane * 0 + j) < cnt
            k = cand_k[j]
            b2 = lax.bitwise_and(lax.shift_right_logical(k, 9), 0x7FF)
            plsc.addupdate_scatter(hist, [b2, lane], cand_e[j], mask=valid)
            return 0

        lax.fori_loop(0, nmax, r2, 0)
        _suffix_sum(hist, _NB)
        b2s, a2 = _search(hist, a1, target, _NB, lane)

        _zero_hist(hist, 520)

        def r3(j, _):
            k = cand_k[j]
            valid = ((lane * 0 + j) < cnt) & (
                lax.bitwise_and(lax.shift_right_logical(k, 9), 0x7FF) == b2s)
            b3 = lax.bitwise_and(k, 0x1FF)
            plsc.addupdate_scatter(hist, [b3, lane], cand_e[j], mask=valid)
            return 0

        lax.fori_loop(0, nmax, r3, 0)
        _suffix_sum(hist, 512)
        b3s, _ = _search(hist, a2, target, 512, lane)

        tau_k = lax.bitwise_or(
            lax.bitwise_or(lax.shift_left(b1s, 20), lax.shift_left(b2s, 9)),
            b3s)
        tau = plsc.bitcast(tau_k, jnp.float32)
        logz = _ilog_poly(s)

        def scan3_chunk(ci, _):
            pltpu.sync_copy(
                x_hbm.at[b, pl.ds(ci * _CH, _CH), pl.ds(col0, _L)], buf)

            def row(i, _):
                v = buf[i]
                e = jnp.exp(v)
                obuf[i] = jnp.where(e >= tau, v - logz, -70.0)
                return 0

            lax.fori_loop(0, _CH, row, 0, unroll=8)
            pltpu.sync_copy(
                obuf, o_hbm.at[b, pl.ds(ci * _CH, _CH), pl.ds(col0, _L)])
            return 0

        lax.fori_loop(0, nchunks, scan3_chunk, 0)
        return 0

    lax.fori_loop(0, (B * C // _L) // _NW, run_unit, 0)


def kernel(logits):
    B, V, C = logits.shape
    mesh = plsc.VectorSubcoreMesh(
        core_axis_name="c", subcore_axis_name="s", num_cores=2, num_subcores=16)
    f = pl.kernel(
        functools.partial(_sc_body),
        out_type=jax.ShapeDtypeStruct((B, V, C), jnp.float32),
        mesh=mesh,
        compiler_params=pltpu.CompilerParams(
            use_tc_tiling_on_sc=False, needs_layout_passes=False),
        scratch_types=[
            pltpu.VMEM((_CH, _L), jnp.float32),
            pltpu.VMEM((_CH, _L), jnp.float32),
            pltpu.VMEM((_HPAD, _L), jnp.float32),
            pltpu.VMEM((_NCAND + 8, _L), jnp.float32),
            pltpu.VMEM((_NCAND + 8, _L), jnp.int32),
        ],
    )
    return f(logits)


# speedup vs baseline: 1.0836x; 1.0836x over previous
import functools

import jax
import jax.numpy as jnp
from jax import lax
from jax.experimental import pallas as pl
from jax.experimental.pallas import tpu as pltpu
from jax.experimental.pallas import tpu_sc as plsc

_R = 0.86
_L = 16
_NW = 32
_CH = 512
_NB = 2048
_HPAD = 2056
_NCAND = 1024
_LN2 = 0.6931471805599453


def _ilog_poly(s):
    bits = plsc.bitcast(s, jnp.int32)
    ex = lax.shift_right_logical(bits, 23) - 127
    mb = lax.bitwise_or(lax.bitwise_and(bits, 0x7FFFFF), 0x3F800000)
    m = plsc.bitcast(mb, jnp.float32)
    z = (m - 1.0) / (m + 1.0)
    z2 = z * z
    p = 1.0 / 9.0 + z2 * 0.0
    p = 1.0 / 7.0 + z2 * p
    p = 1.0 / 5.0 + z2 * p
    p = 1.0 / 3.0 + z2 * p
    p = 1.0 + z2 * p
    return ex.astype(jnp.float32) * _LN2 + 2.0 * z * p


def _search(hist, a0, target, nbins, lane):
    lo = jnp.zeros((_L,), jnp.int32)
    hi = jnp.full((_L,), nbins - 1, jnp.int32)
    steps = nbins.bit_length() - 1

    def step(_, carry):
        lo, hi = carry
        mid = lax.shift_right_logical(lo + hi + 1, 1)
        sv = plsc.load_gather(hist, [mid, lane])
        c = (a0 + sv) >= target
        return jnp.where(c, mid, lo), jnp.where(c, hi, mid - 1)

    lo, hi = lax.fori_loop(0, steps, step, (lo, hi))
    anext = a0 + plsc.load_gather(hist, [lo + 1, lane])
    return lo, anext


def _zero_hist(hist, nrows):
    def z(j, _):
        hist[j] = jnp.zeros((_L,), jnp.float32)
        return 0

    lax.fori_loop(0, nrows, z, 0, unroll=8)


def _suffix_sum(hist, nbins, hist2=None):
    def sfx(j, acc):
        jr = nbins - 1 - j
        acc = acc + (hist[jr] if hist2 is None else hist[jr] + hist2[jr])
        hist[jr] = acc
        return acc

    return lax.fori_loop(0, nbins, sfx, jnp.zeros((_L,), jnp.float32),
                         unroll=4)


def _sc_body(x_hbm, o_hbm, buf, obuf, hist, hist2, cand_e, cand_k):
    wid = lax.axis_index("s") * 2 + lax.axis_index("c")
    lane = lax.iota(jnp.int32, _L)
    B, V, C = 64, 8192, 32
    nchunks = V // _CH

    def run_unit(t, _):
        unit = t * _NW + wid
        b = lax.shift_right_logical(unit, 1)
        h = lax.bitwise_and(unit, 1)
        col0 = h * _L

        _zero_hist(hist, _HPAD)
        _zero_hist(hist2, _HPAD)

        def scan1_chunk(ci, _):
            pltpu.sync_copy(
                x_hbm.at[b, pl.ds(ci * _CH, _CH), pl.ds(col0, _L)], buf)

            def rowpair(i, _):
                e0 = jnp.exp(buf[2 * i])
                e1 = jnp.exp(buf[2 * i + 1])
                k0 = plsc.bitcast(e0, jnp.int32)
                k1 = plsc.bitcast(e1, jnp.int32)
                plsc.addupdate_scatter(
                    hist, [lax.shift_right_logical(k0, 20), lane], e0)
                plsc.addupdate_scatter(
                    hist2, [lax.shift_right_logical(k1, 20), lane], e1)
                return 0

            return lax.fori_loop(0, _CH // 2, rowpair, 0, unroll=4)

        lax.fori_loop(0, nchunks, scan1_chunk, 0)

        s = _suffix_sum(hist, _NB, hist2)
        target = _R * s
        b1s, a1 = _search(hist, jnp.zeros((_L,), jnp.float32), target, _NB, lane)

        def scan2_chunk(ci, cnt):
            pltpu.sync_copy(
                x_hbm.at[b, pl.ds(ci * _CH, _CH), pl.ds(col0, _L)], buf)

            def row(i, cnt):
                e = jnp.exp(buf[i])
                k = plsc.bitcast(e, jnp.int32)
                m = (lax.shift_right_logical(k, 20) == b1s) & (cnt < _NCAND)
                plsc.store_scatter(cand_e, [cnt, lane], e, mask=m)
                plsc.store_scatter(cand_k, [cnt, lane], k, mask=m)
                return cnt + jnp.where(m, 1, 0)

            return lax.fori_loop(0, _CH, row, cnt, unroll=8)

        cnt = lax.fori_loop(0, nchunks, scan2_chunk, jnp.zeros((_L,), jnp.int32))

        nmax = jnp.max(cnt)

        _zero_hist(hist, _HPAD)

        def r2(j, _):
            valid = (lane * 0 + j) < cnt
            k = cand_k[j]
            b2 = lax.bitwise_and(lax.shift_right_logical(k, 9), 0x7FF)
            plsc.addupdate_scatter(hist, [b2, lane], cand_e[j], mask=valid)
            return 0

        lax.fori_loop(0, nmax, r2, 0)
        _suffix_sum(hist, _NB)
        b2s, a2 = _search(hist, a1, target, _NB, lane)

        _zero_hist(hist, 520)

        def r3(j, _):
            k = cand_k[j]
            valid = ((lane * 0 + j) < cnt) & (
                lax.bitwise_and(lax.shift_right_logical(k, 9), 0x7FF) == b2s)
            b3 = lax.bitwise_and(k, 0x1FF)
            plsc.addupdate_scatter(hist, [b3, lane], cand_e[j], mask=valid)
            return 0

        lax.fori_loop(0, nmax, r3, 0)
        _suffix_sum(hist, 512)
        b3s, _ = _search(hist, a2, target, 512, lane)

        tau_k = lax.bitwise_or(
            lax.bitwise_or(lax.shift_left(b1s, 20), lax.shift_left(b2s, 9)),
            b3s)
        tau = plsc.bitcast(tau_k, jnp.float32)
        logz = _ilog_poly(s)

        def scan3_chunk(ci, _):
            pltpu.sync_copy(
                x_hbm.at[b, pl.ds(ci * _CH, _CH), pl.ds(col0, _L)], buf)

            def row(i, _):
                v = buf[i]
                e = jnp.exp(v)
                obuf[i] = jnp.where(e >= tau, v - logz, -70.0)
                return 0

            lax.fori_loop(0, _CH, row, 0, unroll=8)
            pltpu.sync_copy(
                obuf, o_hbm.at[b, pl.ds(ci * _CH, _CH), pl.ds(col0, _L)])
            return 0

        lax.fori_loop(0, nchunks, scan3_chunk, 0)
        return 0

    lax.fori_loop(0, (B * C // _L) // _NW, run_unit, 0)


def kernel(logits):
    B, V, C = logits.shape
    mesh = plsc.VectorSubcoreMesh(
        core_axis_name="c", subcore_axis_name="s", num_cores=2, num_subcores=16)
    f = pl.kernel(
        functools.partial(_sc_body),
        out_type=jax.ShapeDtypeStruct((B, V, C), jnp.float32),
        mesh=mesh,
        compiler_params=pltpu.CompilerParams(
            use_tc_tiling_on_sc=False, needs_layout_passes=False),
        scratch_types=[
            pltpu.VMEM((_CH, _L), jnp.float32),
            pltpu.VMEM((_CH, _L), jnp.float32),
            pltpu.VMEM((_HPAD, _L), jnp.float32),
            pltpu.VMEM((_HPAD, _L), jnp.float32),
            pltpu.VMEM((_NCAND + 8, _L), jnp.float32),
            pltpu.VMEM((_NCAND + 8, _L), jnp.int32),
        ],
    )
    return f(logits)


# speedup vs baseline: 1.5142x; 1.3974x over previous
import functools

import jax
import jax.numpy as jnp
from jax import lax
from jax.experimental import pallas as pl
from jax.experimental.pallas import tpu as pltpu
from jax.experimental.pallas import tpu_sc as plsc

_R = 0.86
_L = 16
_NW = 32
_CH = 512
_NB = 2048
_HPAD = 2052
_NCAND = 520
_LN2 = 0.6931471805599453


def _ilog_poly(s):
    bits = plsc.bitcast(s, jnp.int32)
    ex = lax.shift_right_logical(bits, 23) - 127
    mb = lax.bitwise_or(lax.bitwise_and(bits, 0x7FFFFF), 0x3F800000)
    m = plsc.bitcast(mb, jnp.float32)
    z = (m - 1.0) / (m + 1.0)
    z2 = z * z
    p = 1.0 / 9.0 + z2 * 0.0
    p = 1.0 / 7.0 + z2 * p
    p = 1.0 / 5.0 + z2 * p
    p = 1.0 / 3.0 + z2 * p
    p = 1.0 + z2 * p
    return ex.astype(jnp.float32) * _LN2 + 2.0 * z * p


def _search(hist, col, a0, target, nbins):
    lo = jnp.zeros((_L,), jnp.int32)
    hi = jnp.full((_L,), nbins - 1, jnp.int32)
    steps = nbins.bit_length() - 1

    def step(_, carry):
        lo, hi = carry
        mid = lax.shift_right_logical(lo + hi + 1, 1)
        sv = plsc.load_gather(hist, [mid, col])
        c = (a0 + sv) >= target
        return jnp.where(c, mid, lo), jnp.where(c, hi, mid - 1)

    lo, hi = lax.fori_loop(0, steps, step, (lo, hi))
    anext = a0 + plsc.load_gather(hist, [lo + 1, col])
    return lo, anext


def _zero_hist2(hist, nrows):
    def z(j, _):
        hist[j, pl.ds(0, _L)] = jnp.zeros((_L,), jnp.float32)
        hist[j, pl.ds(_L, _L)] = jnp.zeros((_L,), jnp.float32)
        return 0

    lax.fori_loop(0, nrows, z, 0, unroll=8)


def _zero_hist1(hist, g, nrows):
    def z(j, _):
        hist[j, pl.ds(g * _L, _L)] = jnp.zeros((_L,), jnp.float32)
        return 0

    lax.fori_loop(0, nrows, z, 0, unroll=8)


def _suffix_sum2(hist, nbins):
    def sfx(j, accs):
        a0, a1 = accs
        jr = nbins - 1 - j
        a0 = a0 + hist[jr, pl.ds(0, _L)]
        a1 = a1 + hist[jr, pl.ds(_L, _L)]
        hist[jr, pl.ds(0, _L)] = a0
        hist[jr, pl.ds(_L, _L)] = a1
        return a0, a1

    zero = jnp.zeros((_L,), jnp.float32)
    return lax.fori_loop(0, nbins, sfx, (zero, zero), unroll=4)


def _suffix_sum1(hist, g, nbins):
    def sfx(j, acc):
        jr = nbins - 1 - j
        acc = acc + hist[jr, pl.ds(g * _L, _L)]
        hist[jr, pl.ds(g * _L, _L)] = acc
        return acc

    return lax.fori_loop(0, nbins, sfx, jnp.zeros((_L,), jnp.float32),
                         unroll=4)


def _sc_body(x_hbm, o_hbm, buf, hist, cand_e, cand_k):
    wid = lax.axis_index("s") * 2 + lax.axis_index("c")
    lane = lax.iota(jnp.int32, _L)
    B, V, C = 64, 8192, 32
    nchunks = V // _CH

    def run_unit(t, _):
        bidx = t * _NW + wid

        _zero_hist2(hist, _HPAD)

        def scan1_chunk(ci, _):
            pltpu.sync_copy(x_hbm.at[bidx, pl.ds(ci * _CH, _CH)], buf)

            def row(i, _):
                e0 = jnp.exp(buf[i, pl.ds(0, _L)])
                e1 = jnp.exp(buf[i, pl.ds(_L, _L)])
                k0 = plsc.bitcast(e0, jnp.int32)
                k1 = plsc.bitcast(e1, jnp.int32)
                plsc.addupdate_scatter(
                    hist, [lax.shift_right_logical(k0, 20), lane], e0)
                plsc.addupdate_scatter(
                    hist, [lax.shift_right_logical(k1, 20), lane + _L], e1)
                return 0

            return lax.fori_loop(0, _CH, row, 0, unroll=8)

        lax.fori_loop(0, nchunks, scan1_chunk, 0)

        s0, s1 = _suffix_sum2(hist, _NB)
        tg0, tg1 = _R * s0, _R * s1
        zero = jnp.zeros((_L,), jnp.float32)
        b1s0, a10 = _search(hist, lane, zero, tg0, _NB)
        b1s1, a11 = _search(hist, lane + _L, zero, tg1, _NB)

        def scan2_chunk(ci, cnts):
            pltpu.sync_copy(x_hbm.at[bidx, pl.ds(ci * _CH, _CH)], buf)

            def row(i, cnts):
                c0, c1 = cnts
                e0 = jnp.exp(buf[i, pl.ds(0, _L)])
                e1 = jnp.exp(buf[i, pl.ds(_L, _L)])
                k0 = plsc.bitcast(e0, jnp.int32)
                k1 = plsc.bitcast(e1, jnp.int32)
                m0 = (lax.shift_right_logical(k0, 20) == b1s0) & (c0 < _NCAND)
                m1 = (lax.shift_right_logical(k1, 20) == b1s1) & (c1 < _NCAND)
                plsc.store_scatter(cand_e, [c0, lane], e0, mask=m0)
                plsc.store_scatter(cand_k, [c0, lane], k0, mask=m0)
                plsc.store_scatter(cand_e, [c1, lane + _L], e1, mask=m1)
                plsc.store_scatter(cand_k, [c1, lane + _L], k1, mask=m1)
                return (c0 + jnp.where(m0, 1, 0), c1 + jnp.where(m1, 1, 0))

            return lax.fori_loop(0, _CH, row, cnts, unroll=8)

        czero = jnp.zeros((_L,), jnp.int32)
        cnt0, cnt1 = lax.fori_loop(0, nchunks, scan2_chunk, (czero, czero))

        taus = []
        for g, (b1s, a1, cnt, tgt) in enumerate(
                ((b1s0, a10, cnt0, tg0), (b1s1, a11, cnt1, tg1))):
            col = lane + g * _L
            nmax = jnp.max(cnt)

            _zero_hist1(hist, g, 1028)

            def r2(j, _, cnt=cnt, col=col, g=g):
                valid = (lane * 0 + j) < cnt
                k = cand_k[j, pl.ds(g * _L, _L)]
                b2 = lax.bitwise_and(lax.shift_right_logical(k, 10), 0x3FF)
                plsc.addupdate_scatter(
                    hist, [b2, col], cand_e[j, pl.ds(g * _L, _L)], mask=valid)
                return 0

            lax.fori_loop(0, nmax, r2, 0)
            _suffix_sum1(hist, g, 1024)
            b2s, a2 = _search(hist, col, a1, tgt, 1024)

            _zero_hist1(hist, g, 1028)

            def r3(j, _, cnt=cnt, col=col, g=g, b2s=b2s):
                k = cand_k[j, pl.ds(g * _L, _L)]
                valid = ((lane * 0 + j) < cnt) & (
                    lax.bitwise_and(lax.shift_right_logical(k, 10), 0x3FF)
                    == b2s)
                b3 = lax.bitwise_and(k, 0x3FF)
                plsc.addupdate_scatter(
                    hist, [b3, col], cand_e[j, pl.ds(g * _L, _L)], mask=valid)
                return 0

            lax.fori_loop(0, nmax, r3, 0)
            _suffix_sum1(hist, g, 1024)
            b3s, _ = _search(hist, col, a2, tgt, 1024)

            tau_k = lax.bitwise_or(
                lax.bitwise_or(lax.shift_left(b1s, 20),
                               lax.shift_left(b2s, 10)), b3s)
            taus.append(plsc.bitcast(tau_k, jnp.float32))

        tau0, tau1 = taus
        logz0 = _ilog_poly(s0)
        logz1 = _ilog_poly(s1)

        def scan3_chunk(ci, _):
            pltpu.sync_copy(x_hbm.at[bidx, pl.ds(ci * _CH, _CH)], buf)

            def row(i, _):
                v0 = buf[i, pl.ds(0, _L)]
                v1 = buf[i, pl.ds(_L, _L)]
                e0 = jnp.exp(v0)
                e1 = jnp.exp(v1)
                cand_e[i, pl.ds(0, _L)] = jnp.where(
                    e0 >= tau0, v0 - logz0, -70.0)
                cand_e[i, pl.ds(_L, _L)] = jnp.where(
                    e1 >= tau1, v1 - logz1, -70.0)
                return 0

            lax.fori_loop(0, _CH, row, 0, unroll=8)
            pltpu.sync_copy(cand_e.at[pl.ds(0, _CH)],
                            o_hbm.at[bidx, pl.ds(ci * _CH, _CH)])
            return 0

        lax.fori_loop(0, nchunks, scan3_chunk, 0)
        return 0

    lax.fori_loop(0, B // _NW, run_unit, 0)


def kernel(logits):
    B, V, C = logits.shape
    mesh = plsc.VectorSubcoreMesh(
        core_axis_name="c", subcore_axis_name="s", num_cores=2, num_subcores=16)
    f = pl.kernel(
        functools.partial(_sc_body),
        out_type=jax.ShapeDtypeStruct((B, V, C), jnp.float32),
        mesh=mesh,
        compiler_params=pltpu.CompilerParams(
            use_tc_tiling_on_sc=False, needs_layout_passes=False),
        scratch_types=[
            pltpu.VMEM((_CH, C), jnp.float32),
            pltpu.VMEM((_HPAD, C), jnp.float32),
            pltpu.VMEM((_NCAND, C), jnp.float32),
            pltpu.VMEM((_NCAND, C), jnp.int32),
        ],
    )
    return f(logits)


# speedup vs baseline: 1.7834x; 1.1778x over previous
import functools

import jax
import jax.numpy as jnp
from jax import lax
from jax.experimental import pallas as pl
from jax.experimental.pallas import tpu as pltpu
from jax.experimental.pallas import tpu_sc as plsc

_R = 0.86
_L = 16
_NW = 32
_CH = 512
_NB = 2048
_HPAD = 2052
_NCAND = 520
_LN2 = 0.6931471805599453


def _ilog_poly(s):
    bits = plsc.bitcast(s, jnp.int32)
    ex = lax.shift_right_logical(bits, 23) - 127
    mb = lax.bitwise_or(lax.bitwise_and(bits, 0x7FFFFF), 0x3F800000)
    m = plsc.bitcast(mb, jnp.float32)
    z = (m - 1.0) / (m + 1.0)
    z2 = z * z
    p = 1.0 / 9.0 + z2 * 0.0
    p = 1.0 / 7.0 + z2 * p
    p = 1.0 / 5.0 + z2 * p
    p = 1.0 / 3.0 + z2 * p
    p = 1.0 + z2 * p
    return ex.astype(jnp.float32) * _LN2 + 2.0 * z * p


def _search(hist, col, a0, target, nbins):
    lo = jnp.zeros((_L,), jnp.int32)
    hi = jnp.full((_L,), nbins - 1, jnp.int32)
    steps = nbins.bit_length() - 1

    def step(_, carry):
        lo, hi = carry
        mid = lax.shift_right_logical(lo + hi + 1, 1)
        sv = plsc.load_gather(hist, [mid, col])
        c = (a0 + sv) >= target
        return jnp.where(c, mid, lo), jnp.where(c, hi, mid - 1)

    lo, hi = lax.fori_loop(0, steps, step, (lo, hi))
    anext = a0 + plsc.load_gather(hist, [lo + 1, col])
    return lo, anext


def _zero_hist2(hist, nrows):
    def z(j, _):
        hist[j, pl.ds(0, _L)] = jnp.zeros((_L,), jnp.float32)
        hist[j, pl.ds(_L, _L)] = jnp.zeros((_L,), jnp.float32)
        return 0

    lax.fori_loop(0, nrows, z, 0, unroll=8)


def _zero_hist1(hist, g, nrows):
    def z(j, _):
        hist[j, pl.ds(g * _L, _L)] = jnp.zeros((_L,), jnp.float32)
        return 0

    lax.fori_loop(0, nrows, z, 0, unroll=8)


def _suffix_sum2(hist, nbins):
    def sfx(j, accs):
        a0, a1 = accs
        jr = nbins - 1 - j
        a0 = a0 + hist[jr, pl.ds(0, _L)]
        a1 = a1 + hist[jr, pl.ds(_L, _L)]
        hist[jr, pl.ds(0, _L)] = a0
        hist[jr, pl.ds(_L, _L)] = a1
        return a0, a1

    zero = jnp.zeros((_L,), jnp.float32)
    return lax.fori_loop(0, nbins, sfx, (zero, zero), unroll=4)


def _suffix_sum1(hist, g, nbins):
    def sfx(j, acc):
        jr = nbins - 1 - j
        acc = acc + hist[jr, pl.ds(g * _L, _L)]
        hist[jr, pl.ds(g * _L, _L)] = acc
        return acc

    return lax.fori_loop(0, nbins, sfx, jnp.zeros((_L,), jnp.float32),
                         unroll=4)


def _sc_body(x_hbm, o_hbm, buf, hist, cand_e, cand_k):
    wid = lax.axis_index("s") * 2 + lax.axis_index("c")
    lane = lax.iota(jnp.int32, _L)
    B, V, C = x_hbm.shape
    nchunks = V // _CH

    def run_unit(t, _):
        bidx = t * _NW + wid

        _zero_hist2(hist, _HPAD)

        def scan1_chunk(ci, _):
            pltpu.sync_copy(x_hbm.at[bidx, pl.ds(ci * _CH, _CH)], buf)

            def row(i, _):
                e0 = jnp.exp(buf[i, pl.ds(0, _L)])
                e1 = jnp.exp(buf[i, pl.ds(_L, _L)])
                k0 = plsc.bitcast(e0, jnp.int32)
                k1 = plsc.bitcast(e1, jnp.int32)
                plsc.addupdate_scatter(
                    hist, [lax.shift_right_logical(k0, 20), lane], e0)
                plsc.addupdate_scatter(
                    hist, [lax.shift_right_logical(k1, 20), lane + _L], e1)
                return 0

            return lax.fori_loop(0, _CH, row, 0, unroll=8)

        lax.fori_loop(0, nchunks, scan1_chunk, 0)

        s0, s1 = _suffix_sum2(hist, _NB)
        tg0, tg1 = _R * s0, _R * s1
        zero = jnp.zeros((_L,), jnp.float32)
        b1s0, a10 = _search(hist, lane, zero, tg0, _NB)
        b1s1, a11 = _search(hist, lane + _L, zero, tg1, _NB)

        def scan2_chunk(ci, cnts):
            pltpu.sync_copy(x_hbm.at[bidx, pl.ds(ci * _CH, _CH)], buf)

            def row(i, cnts):
                c0, c1 = cnts
                e0 = jnp.exp(buf[i, pl.ds(0, _L)])
                e1 = jnp.exp(buf[i, pl.ds(_L, _L)])
                k0 = plsc.bitcast(e0, jnp.int32)
                k1 = plsc.bitcast(e1, jnp.int32)
                m0 = (lax.shift_right_logical(k0, 20) == b1s0) & (c0 < _NCAND)
                m1 = (lax.shift_right_logical(k1, 20) == b1s1) & (c1 < _NCAND)
                plsc.store_scatter(cand_e, [c0, lane], e0, mask=m0)
                plsc.store_scatter(cand_k, [c0, lane], k0, mask=m0)
                plsc.store_scatter(cand_e, [c1, lane + _L], e1, mask=m1)
                plsc.store_scatter(cand_k, [c1, lane + _L], k1, mask=m1)
                return (c0 + jnp.where(m0, 1, 0), c1 + jnp.where(m1, 1, 0))

            return lax.fori_loop(0, _CH, row, cnts, unroll=8)

        czero = jnp.zeros((_L,), jnp.int32)
        cnt0, cnt1 = lax.fori_loop(0, nchunks, scan2_chunk, (czero, czero))

        taus = []
        for g, (b1s, a1, cnt, tgt) in enumerate(
                ((b1s0, a10, cnt0, tg0), (b1s1, a11, cnt1, tg1))):
            col = lane + g * _L
            nmax = jnp.max(cnt)

            _zero_hist1(hist, g, 1028)

            def r2(j, _, cnt=cnt, col=col, g=g):
                valid = (lane * 0 + j) < cnt
                k = cand_k[j, pl.ds(g * _L, _L)]
                b2 = lax.bitwise_and(lax.shift_right_logical(k, 10), 0x3FF)
                plsc.addupdate_scatter(
                    hist, [b2, col], cand_e[j, pl.ds(g * _L, _L)], mask=valid)
                return 0

            lax.fori_loop(0, nmax, r2, 0)
            _suffix_sum1(hist, g, 1024)
            b2s, a2 = _search(hist, col, a1, tgt, 1024)

            _zero_hist1(hist, g, 1028)

            def r3(j, _, cnt=cnt, col=col, g=g, b2s=b2s):
                k = cand_k[j, pl.ds(g * _L, _L)]
                valid = ((lane * 0 + j) < cnt) & (
                    lax.bitwise_and(lax.shift_right_logical(k, 10), 0x3FF)
                    == b2s)
                b3 = lax.bitwise_and(k, 0x3FF)
                plsc.addupdate_scatter(
                    hist, [b3, col], cand_e[j, pl.ds(g * _L, _L)], mask=valid)
                return 0

            lax.fori_loop(0, nmax, r3, 0)
            _suffix_sum1(hist, g, 1024)
            b3s, _ = _search(hist, col, a2, tgt, 1024)

            tau_k = lax.bitwise_or(
                lax.bitwise_or(lax.shift_left(b1s, 20),
                               lax.shift_left(b2s, 10)), b3s)
            taus.append(plsc.bitcast(tau_k, jnp.float32))

        tau0, tau1 = taus
        logz0 = _ilog_poly(s0)
        logz1 = _ilog_poly(s1)

        def scan3_chunk(ci, _):
            pltpu.sync_copy(x_hbm.at[bidx, pl.ds(ci * _CH, _CH)], buf)

            def row(i, _):
                v0 = buf[i, pl.ds(0, _L)]
                v1 = buf[i, pl.ds(_L, _L)]
                e0 = jnp.exp(v0)
                e1 = jnp.exp(v1)
                cand_e[i, pl.ds(0, _L)] = jnp.where(
                    e0 >= tau0, v0 - logz0, -70.0)
                cand_e[i, pl.ds(_L, _L)] = jnp.where(
                    e1 >= tau1, v1 - logz1, -70.0)
                return 0

            lax.fori_loop(0, _CH, row, 0, unroll=8)
            pltpu.sync_copy(cand_e.at[pl.ds(0, _CH)],
                            o_hbm.at[bidx, pl.ds(ci * _CH, _CH)])
            return 0

        lax.fori_loop(0, nchunks, scan3_chunk, 0)
        return 0

    lax.fori_loop(0, B // _NW, run_unit, 0)


def _sc_part(x):
    B, V, C = x.shape
    mesh = plsc.VectorSubcoreMesh(
        core_axis_name="c", subcore_axis_name="s", num_cores=2, num_subcores=16)
    f = pl.kernel(
        functools.partial(_sc_body),
        out_type=jax.ShapeDtypeStruct((B, V, C), jnp.float32),
        mesh=mesh,
        compiler_params=pltpu.CompilerParams(
            use_tc_tiling_on_sc=False, needs_layout_passes=False),
        scratch_types=[
            pltpu.VMEM((_CH, C), jnp.float32),
            pltpu.VMEM((_HPAD, C), jnp.float32),
            pltpu.VMEM((_NCAND, C), jnp.float32),
            pltpu.VMEM((_NCAND, C), jnp.int32),
        ],
    )
    return f(x)


def _tc_body(x_ref, o_ref):
    x = x_ref[...]
    m = jnp.max(x, axis=0, keepdims=True)
    shifted = x - m
    e = jnp.exp(shifted)
    s = jnp.sum(e, axis=0, keepdims=True)
    target = _R * s

    lo = jnp.zeros(s.shape, jnp.int32)
    hi = jnp.full(s.shape, 0x3F800000, jnp.int32)

    def step(_, carry):
        lo, hi = carry
        mid = lo + ((hi - lo) >> 1)
        mid_f = jax.lax.bitcast_convert_type(mid, jnp.float32)
        g = jnp.sum(jnp.where(e > mid_f, e, 0.0), axis=0, keepdims=True)
        cond = g < target
        return jnp.where(cond, lo, mid + 1), jnp.where(cond, mid, hi)

    lo, hi = lax.fori_loop(0, 30, step, (lo, hi))
    thr = jax.lax.bitcast_convert_type(lo, jnp.float32)
    o_ref[...] = jnp.where(e >= thr, shifted - jnp.log(s), -70.0)


def _tc_part(x):
    B, V, C = x.shape
    xt = x.transpose(1, 0, 2).reshape(V, B * C)
    out = pl.pallas_call(
        _tc_body,
        grid=(B * C // 128,),
        in_specs=[pl.BlockSpec((V, 128), lambda c: (0, c))],
        out_specs=pl.BlockSpec((V, 128), lambda c: (0, c)),
        out_shape=jax.ShapeDtypeStruct((V, B * C), jnp.float32),
    )(xt)
    return out.reshape(V, B, C).transpose(1, 0, 2)


_B_SC = 32


def kernel(logits):
    out_sc = _sc_part(logits[:_B_SC])
    out_tc = _tc_part(logits[_B_SC:])
    return jnp.concatenate([out_sc, out_tc], axis=0)


# speedup vs baseline: 2.3539x; 1.3199x over previous
import functools

import jax
import jax.numpy as jnp
from jax import lax
from jax.experimental import pallas as pl
from jax.experimental.pallas import tpu as pltpu
from jax.experimental.pallas import tpu_sc as plsc

_R = 0.86
_L = 16
_NW = 32
_CH = 512
_NB = 2048
_HPAD = 2052
_NCAND = 520
_LN2 = 0.6931471805599453


def _ilog_poly(s):
    bits = plsc.bitcast(s, jnp.int32)
    ex = lax.shift_right_logical(bits, 23) - 127
    mb = lax.bitwise_or(lax.bitwise_and(bits, 0x7FFFFF), 0x3F800000)
    m = plsc.bitcast(mb, jnp.float32)
    z = (m - 1.0) / (m + 1.0)
    z2 = z * z
    p = 1.0 / 9.0 + z2 * 0.0
    p = 1.0 / 7.0 + z2 * p
    p = 1.0 / 5.0 + z2 * p
    p = 1.0 / 3.0 + z2 * p
    p = 1.0 + z2 * p
    return ex.astype(jnp.float32) * _LN2 + 2.0 * z * p


def _search(hist, col, a0, target, nbins):
    lo = jnp.zeros((_L,), jnp.int32)
    hi = jnp.full((_L,), nbins - 1, jnp.int32)
    steps = nbins.bit_length() - 1

    def step(_, carry):
        lo, hi = carry
        mid = lax.shift_right_logical(lo + hi + 1, 1)
        sv = plsc.load_gather(hist, [mid, col])
        c = (a0 + sv) >= target
        return jnp.where(c, mid, lo), jnp.where(c, hi, mid - 1)

    lo, hi = lax.fori_loop(0, steps, step, (lo, hi))
    anext = a0 + plsc.load_gather(hist, [lo + 1, col])
    return lo, anext


def _zero_hist2(hist, nrows):
    def z(j, _):
        hist[j, pl.ds(0, _L)] = jnp.zeros((_L,), jnp.float32)
        hist[j, pl.ds(_L, _L)] = jnp.zeros((_L,), jnp.float32)
        return 0

    lax.fori_loop(0, nrows, z, 0, unroll=8)


def _zero_hist1(hist, g, nrows):
    def z(j, _):
        hist[j, pl.ds(g * _L, _L)] = jnp.zeros((_L,), jnp.float32)
        return 0

    lax.fori_loop(0, nrows, z, 0, unroll=8)


def _suffix_sum2(hist, nbins):
    def sfx(j, accs):
        a0, a1 = accs
        jr = nbins - 1 - j
        a0 = a0 + hist[jr, pl.ds(0, _L)]
        a1 = a1 + hist[jr, pl.ds(_L, _L)]
        hist[jr, pl.ds(0, _L)] = a0
        hist[jr, pl.ds(_L, _L)] = a1
        return a0, a1

    zero = jnp.zeros((_L,), jnp.float32)
    return lax.fori_loop(0, nbins, sfx, (zero, zero), unroll=4)


def _suffix_sum1(hist, g, nbins):
    def sfx(j, acc):
        jr = nbins - 1 - j
        acc = acc + hist[jr, pl.ds(g * _L, _L)]
        hist[jr, pl.ds(g * _L, _L)] = acc
        return acc

    return lax.fori_loop(0, nbins, sfx, jnp.zeros((_L,), jnp.float32),
                         unroll=4)


def _sc_body(x_hbm, o_hbm, buf, hist, cand_e, cand_k):
    wid = lax.axis_index("s") * 2 + lax.axis_index("c")
    lane = lax.iota(jnp.int32, _L)
    B, V, C = x_hbm.shape
    nchunks = V // _CH

    def run_unit(t, _):
        bidx = t * _NW + wid

        _zero_hist2(hist, _HPAD)

        def scan1_chunk(ci, _):
            pltpu.sync_copy(x_hbm.at[bidx, pl.ds(ci * _CH, _CH)], buf)

            @plsc.parallel_loop(0, _CH, unroll=8)
            def row(i):
                e0 = jnp.exp(buf[i, pl.ds(0, _L)])
                e1 = jnp.exp(buf[i, pl.ds(_L, _L)])
                k0 = plsc.bitcast(e0, jnp.int32)
                k1 = plsc.bitcast(e1, jnp.int32)
                plsc.addupdate_scatter(
                    hist, [lax.shift_right_logical(k0, 20), lane], e0)
                plsc.addupdate_scatter(
                    hist, [lax.shift_right_logical(k1, 20), lane + _L], e1)

            return 0

        lax.fori_loop(0, nchunks, scan1_chunk, 0)

        s0, s1 = _suffix_sum2(hist, _NB)
        tg0, tg1 = _R * s0, _R * s1
        zero = jnp.zeros((_L,), jnp.float32)
        b1s0, a10 = _search(hist, lane, zero, tg0, _NB)
        b1s1, a11 = _search(hist, lane + _L, zero, tg1, _NB)

        def scan2_chunk(ci, cnts):
            pltpu.sync_copy(x_hbm.at[bidx, pl.ds(ci * _CH, _CH)], buf)

            @plsc.parallel_loop(0, _CH, unroll=8, carry=cnts)
            def row(i, cnts):
                c0, c1 = cnts
                e0 = jnp.exp(buf[i, pl.ds(0, _L)])
                e1 = jnp.exp(buf[i, pl.ds(_L, _L)])
                k0 = plsc.bitcast(e0, jnp.int32)
                k1 = plsc.bitcast(e1, jnp.int32)
                m0 = (lax.shift_right_logical(k0, 20) == b1s0) & (c0 < _NCAND)
                m1 = (lax.shift_right_logical(k1, 20) == b1s1) & (c1 < _NCAND)
                plsc.store_scatter(cand_e, [c0, lane], e0, mask=m0)
                plsc.store_scatter(cand_k, [c0, lane], k0, mask=m0)
                plsc.store_scatter(cand_e, [c1, lane + _L], e1, mask=m1)
                plsc.store_scatter(cand_k, [c1, lane + _L], k1, mask=m1)
                return (c0 + jnp.where(m0, 1, 0), c1 + jnp.where(m1, 1, 0))

            return row

        czero = jnp.zeros((_L,), jnp.int32)
        cnt0, cnt1 = lax.fori_loop(0, nchunks, scan2_chunk, (czero, czero))

        taus = []
        for g, (b1s, a1, cnt, tgt) in enumerate(
                ((b1s0, a10, cnt0, tg0), (b1s1, a11, cnt1, tg1))):
            col = lane + g * _L
            nmax = jnp.max(cnt)

            _zero_hist1(hist, g, 1028)

            def r2(j, _, cnt=cnt, col=col, g=g):
                valid = (lane * 0 + j) < cnt
                k = cand_k[j, pl.ds(g * _L, _L)]
                b2 = lax.bitwise_and(lax.shift_right_logical(k, 10), 0x3FF)
                plsc.addupdate_scatter(
                    hist, [b2, col], cand_e[j, pl.ds(g * _L, _L)], mask=valid)
                return 0

            lax.fori_loop(0, nmax, r2, 0)
            _suffix_sum1(hist, g, 1024)
            b2s, a2 = _search(hist, col, a1, tgt, 1024)

            _zero_hist1(hist, g, 1028)

            def r3(j, _, cnt=cnt, col=col, g=g, b2s=b2s):
                k = cand_k[j, pl.ds(g * _L, _L)]
                valid = ((lane * 0 + j) < cnt) & (
                    lax.bitwise_and(lax.shift_right_logical(k, 10), 0x3FF)
                    == b2s)
                b3 = lax.bitwise_and(k, 0x3FF)
                plsc.addupdate_scatter(
                    hist, [b3, col], cand_e[j, pl.ds(g * _L, _L)], mask=valid)
                return 0

            lax.fori_loop(0, nmax, r3, 0)
            _suffix_sum1(hist, g, 1024)
            b3s, _ = _search(hist, col, a2, tgt, 1024)

            tau_k = lax.bitwise_or(
                lax.bitwise_or(lax.shift_left(b1s, 20),
                               lax.shift_left(b2s, 10)), b3s)
            taus.append(plsc.bitcast(tau_k, jnp.float32))

        tau0, tau1 = taus
        logz0 = _ilog_poly(s0)
        logz1 = _ilog_poly(s1)

        def scan3_chunk(ci, _):
            pltpu.sync_copy(x_hbm.at[bidx, pl.ds(ci * _CH, _CH)], buf)

            @plsc.parallel_loop(0, _CH, unroll=8)
            def row(i):
                v0 = buf[i, pl.ds(0, _L)]
                v1 = buf[i, pl.ds(_L, _L)]
                e0 = jnp.exp(v0)
                e1 = jnp.exp(v1)
                cand_e[i, pl.ds(0, _L)] = jnp.where(
                    e0 >= tau0, v0 - logz0, -70.0)
                cand_e[i, pl.ds(_L, _L)] = jnp.where(
                    e1 >= tau1, v1 - logz1, -70.0)
            pltpu.sync_copy(cand_e.at[pl.ds(0, _CH)],
                            o_hbm.at[bidx, pl.ds(ci * _CH, _CH)])
            return 0

        lax.fori_loop(0, nchunks, scan3_chunk, 0)
        return 0

    lax.fori_loop(0, B // _NW, run_unit, 0)


def _sc_part(x):
    B, V, C = x.shape
    mesh = plsc.VectorSubcoreMesh(
        core_axis_name="c", subcore_axis_name="s", num_cores=2, num_subcores=16)
    f = pl.kernel(
        functools.partial(_sc_body),
        out_type=jax.ShapeDtypeStruct((B, V, C), jnp.float32),
        mesh=mesh,
        compiler_params=pltpu.CompilerParams(
            use_tc_tiling_on_sc=False, needs_layout_passes=False),
        scratch_types=[
            pltpu.VMEM((_CH, C), jnp.float32),
            pltpu.VMEM((_HPAD, C), jnp.float32),
            pltpu.VMEM((_NCAND, C), jnp.float32),
            pltpu.VMEM((_NCAND, C), jnp.int32),
        ],
    )
    return f(x)


def _tc_body(x_ref, o_ref):
    x = x_ref[...]
    m = jnp.max(x, axis=0, keepdims=True)
    shifted = x - m
    e = jnp.exp(shifted)
    s = jnp.sum(e, axis=0, keepdims=True)
    target = _R * s

    lo = jnp.zeros(s.shape, jnp.int32)
    hi = jnp.full(s.shape, 0x3F800000, jnp.int32)

    def step(_, carry):
        lo, hi = carry
        mid = lo + ((hi - lo) >> 1)
        mid_f = jax.lax.bitcast_convert_type(mid, jnp.float32)
        g = jnp.sum(jnp.where(e > mid_f, e, 0.0), axis=0, keepdims=True)
        cond = g < target
        return jnp.where(cond, lo, mid + 1), jnp.where(cond, mid, hi)

    lo, hi = lax.fori_loop(0, 30, step, (lo, hi))
    thr = jax.lax.bitcast_convert_type(lo, jnp.float32)
    o_ref[...] = jnp.where(e >= thr, shifted - jnp.log(s), -70.0)


def _tc_part(x):
    B, V, C = x.shape
    xt = x.transpose(1, 0, 2).reshape(V, B * C)
    out = pl.pallas_call(
        _tc_body,
        grid=(B * C // 128,),
        in_specs=[pl.BlockSpec((V, 128), lambda c: (0, c))],
        out_specs=pl.BlockSpec((V, 128), lambda c: (0, c)),
        out_shape=jax.ShapeDtypeStruct((V, B * C), jnp.float32),
    )(xt)
    return out.reshape(V, B, C).transpose(1, 0, 2)


_B_SC = 32


def kernel(logits):
    out_sc = _sc_part(logits[:_B_SC])
    out_tc = _tc_part(logits[_B_SC:])
    return jnp.concatenate([out_sc, out_tc], axis=0)


# speedup vs baseline: 2.3540x; 1.0000x over previous
import functools

import jax
import jax.numpy as jnp
from jax import lax
from jax.experimental import pallas as pl
from jax.experimental.pallas import tpu as pltpu
from jax.experimental.pallas import tpu_sc as plsc

_R = 0.86
_L = 16
_NW = 32
_CH = 512
_NB = 2048
_HPAD = 2052
_NCAND = 520
_LN2 = 0.6931471805599453


def _ilog_poly(s):
    bits = plsc.bitcast(s, jnp.int32)
    ex = lax.shift_right_logical(bits, 23) - 127
    mb = lax.bitwise_or(lax.bitwise_and(bits, 0x7FFFFF), 0x3F800000)
    m = plsc.bitcast(mb, jnp.float32)
    z = (m - 1.0) / (m + 1.0)
    z2 = z * z
    p = 1.0 / 9.0 + z2 * 0.0
    p = 1.0 / 7.0 + z2 * p
    p = 1.0 / 5.0 + z2 * p
    p = 1.0 / 3.0 + z2 * p
    p = 1.0 + z2 * p
    return ex.astype(jnp.float32) * _LN2 + 2.0 * z * p


def _search(hist, col, a0, target, nbins):
    lo = jnp.zeros((_L,), jnp.int32)
    hi = jnp.full((_L,), nbins - 1, jnp.int32)
    steps = nbins.bit_length() - 1

    def step(_, carry):
        lo, hi = carry
        mid = lax.shift_right_logical(lo + hi + 1, 1)
        sv = plsc.load_gather(hist, [mid, col])
        c = (a0 + sv) >= target
        return jnp.where(c, mid, lo), jnp.where(c, hi, mid - 1)

    lo, hi = lax.fori_loop(0, steps, step, (lo, hi))
    anext = a0 + plsc.load_gather(hist, [lo + 1, col])
    return lo, anext


def _zero_hist2(hist, nrows):
    @plsc.parallel_loop(0, nrows, unroll=8)
    def z(j):
        hist[j, pl.ds(0, _L)] = jnp.zeros((_L,), jnp.float32)
        hist[j, pl.ds(_L, _L)] = jnp.zeros((_L,), jnp.float32)


def _zero_hist1(hist, g, nrows):
    @plsc.parallel_loop(0, nrows, unroll=8)
    def z(j):
        hist[j, pl.ds(g * _L, _L)] = jnp.zeros((_L,), jnp.float32)


def _suffix_sum2(hist, nbins):
    def sfx(j, accs):
        a0, a1 = accs
        jr = nbins - 1 - j
        a0 = a0 + hist[jr, pl.ds(0, _L)]
        a1 = a1 + hist[jr, pl.ds(_L, _L)]
        hist[jr, pl.ds(0, _L)] = a0
        hist[jr, pl.ds(_L, _L)] = a1
        return a0, a1

    zero = jnp.zeros((_L,), jnp.float32)
    return lax.fori_loop(0, nbins, sfx, (zero, zero), unroll=4)


def _suffix_sum1(hist, g, nbins):
    def sfx(j, acc):
        jr = nbins - 1 - j
        acc = acc + hist[jr, pl.ds(g * _L, _L)]
        hist[jr, pl.ds(g * _L, _L)] = acc
        return acc

    return lax.fori_loop(0, nbins, sfx, jnp.zeros((_L,), jnp.float32),
                         unroll=4)


def _sc_body(x_hbm, o_hbm, buf, hist, cand_e, cand_k):
    wid = lax.axis_index("s") * 2 + lax.axis_index("c")
    lane = lax.iota(jnp.int32, _L)
    B, V, C = x_hbm.shape
    nchunks = V // _CH

    def run_unit(t, _):
        bidx = t * _NW + wid

        _zero_hist2(hist, _HPAD)

        def scan1_chunk(ci, _):
            pltpu.sync_copy(x_hbm.at[bidx, pl.ds(ci * _CH, _CH)], buf)

            @plsc.parallel_loop(0, _CH, unroll=8)
            def row(i):
                e0 = jnp.exp(buf[i, pl.ds(0, _L)])
                e1 = jnp.exp(buf[i, pl.ds(_L, _L)])
                k0 = plsc.bitcast(e0, jnp.int32)
                k1 = plsc.bitcast(e1, jnp.int32)
                plsc.addupdate_scatter(
                    hist, [lax.shift_right_logical(k0, 20), lane], e0)
                plsc.addupdate_scatter(
                    hist, [lax.shift_right_logical(k1, 20), lane + _L], e1)

            return 0

        lax.fori_loop(0, nchunks, scan1_chunk, 0)

        s0, s1 = _suffix_sum2(hist, _NB)
        tg0, tg1 = _R * s0, _R * s1
        zero = jnp.zeros((_L,), jnp.float32)
        b1s0, a10 = _search(hist, lane, zero, tg0, _NB)
        b1s1, a11 = _search(hist, lane + _L, zero, tg1, _NB)

        def scan2_chunk(ci, cnts):
            pltpu.sync_copy(x_hbm.at[bidx, pl.ds(ci * _CH, _CH)], buf)

            @plsc.parallel_loop(0, _CH, unroll=8, carry=cnts)
            def row(i, cnts):
                c0, c1 = cnts
                e0 = jnp.exp(buf[i, pl.ds(0, _L)])
                e1 = jnp.exp(buf[i, pl.ds(_L, _L)])
                k0 = plsc.bitcast(e0, jnp.int32)
                k1 = plsc.bitcast(e1, jnp.int32)
                m0 = (lax.shift_right_logical(k0, 20) == b1s0) & (c0 < _NCAND)
                m1 = (lax.shift_right_logical(k1, 20) == b1s1) & (c1 < _NCAND)
                plsc.store_scatter(cand_e, [c0, lane], e0, mask=m0)
                plsc.store_scatter(cand_k, [c0, lane], k0, mask=m0)
                plsc.store_scatter(cand_e, [c1, lane + _L], e1, mask=m1)
                plsc.store_scatter(cand_k, [c1, lane + _L], k1, mask=m1)
                return (c0 + jnp.where(m0, 1, 0), c1 + jnp.where(m1, 1, 0))

            return row

        czero = jnp.zeros((_L,), jnp.int32)
        cnt0, cnt1 = lax.fori_loop(0, nchunks, scan2_chunk, (czero, czero))

        taus = []
        for g, (b1s, a1, cnt, tgt) in enumerate(
                ((b1s0, a10, cnt0, tg0), (b1s1, a11, cnt1, tg1))):
            col = lane + g * _L
            nmax = jnp.max(cnt)

            _zero_hist1(hist, g, 1028)

            def r2(j, _, cnt=cnt, col=col, g=g):
                valid = (lane * 0 + j) < cnt
                k = cand_k[j, pl.ds(g * _L, _L)]
                b2 = lax.bitwise_and(lax.shift_right_logical(k, 10), 0x3FF)
                plsc.addupdate_scatter(
                    hist, [b2, col], cand_e[j, pl.ds(g * _L, _L)], mask=valid)
                return 0

            lax.fori_loop(0, nmax, r2, 0)
            _suffix_sum1(hist, g, 1024)
            b2s, a2 = _search(hist, col, a1, tgt, 1024)

            _zero_hist1(hist, g, 1028)

            def r3(j, _, cnt=cnt, col=col, g=g, b2s=b2s):
                k = cand_k[j, pl.ds(g * _L, _L)]
                valid = ((lane * 0 + j) < cnt) & (
                    lax.bitwise_and(lax.shift_right_logical(k, 10), 0x3FF)
                    == b2s)
                b3 = lax.bitwise_and(k, 0x3FF)
                plsc.addupdate_scatter(
                    hist, [b3, col], cand_e[j, pl.ds(g * _L, _L)], mask=valid)
                return 0

            lax.fori_loop(0, nmax, r3, 0)
            _suffix_sum1(hist, g, 1024)
            b3s, _ = _search(hist, col, a2, tgt, 1024)

            tau_k = lax.bitwise_or(
                lax.bitwise_or(lax.shift_left(b1s, 20),
                               lax.shift_left(b2s, 10)), b3s)
            taus.append(plsc.bitcast(tau_k, jnp.float32))

        tau0, tau1 = taus
        logz0 = _ilog_poly(s0)
        logz1 = _ilog_poly(s1)

        def scan3_chunk(ci, _):
            pltpu.sync_copy(x_hbm.at[bidx, pl.ds(ci * _CH, _CH)], buf)

            @plsc.parallel_loop(0, _CH, unroll=8)
            def row(i):
                v0 = buf[i, pl.ds(0, _L)]
                v1 = buf[i, pl.ds(_L, _L)]
                e0 = jnp.exp(v0)
                e1 = jnp.exp(v1)
                cand_e[i, pl.ds(0, _L)] = jnp.where(
                    e0 >= tau0, v0 - logz0, -70.0)
                cand_e[i, pl.ds(_L, _L)] = jnp.where(
                    e1 >= tau1, v1 - logz1, -70.0)
            pltpu.sync_copy(cand_e.at[pl.ds(0, _CH)],
                            o_hbm.at[bidx, pl.ds(ci * _CH, _CH)])
            return 0

        lax.fori_loop(0, nchunks, scan3_chunk, 0)
        return 0

    lax.fori_loop(0, B // _NW, run_unit, 0)


def _sc_part(x):
    B, V, C = x.shape
    mesh = plsc.VectorSubcoreMesh(
        core_axis_name="c", subcore_axis_name="s", num_cores=2, num_subcores=16)
    f = pl.kernel(
        functools.partial(_sc_body),
        out_type=jax.ShapeDtypeStruct((B, V, C), jnp.float32),
        mesh=mesh,
        compiler_params=pltpu.CompilerParams(
            use_tc_tiling_on_sc=False, needs_layout_passes=False),
        scratch_types=[
            pltpu.VMEM((_CH, C), jnp.float32),
            pltpu.VMEM((_HPAD, C), jnp.float32),
            pltpu.VMEM((_NCAND, C), jnp.float32),
            pltpu.VMEM((_NCAND, C), jnp.int32),
        ],
    )
    return f(x)


def _tc_body(x_ref, o_ref):
    x = x_ref[...]
    m = jnp.max(x, axis=0, keepdims=True)
    shifted = x - m
    e = jnp.exp(shifted)
    s = jnp.sum(e, axis=0, keepdims=True)
    target = _R * s

    lo = jnp.zeros(s.shape, jnp.int32)
    hi = jnp.full(s.shape, 0x3F800000, jnp.int32)

    def step(_, carry):
        lo, hi = carry
        mid = lo + ((hi - lo) >> 1)
        mid_f = jax.lax.bitcast_convert_type(mid, jnp.float32)
        g = jnp.sum(jnp.where(e > mid_f, e, 0.0), axis=0, keepdims=True)
        cond = g < target
        return jnp.where(cond, lo, mid + 1), jnp.where(cond, mid, hi)

    lo, hi = lax.fori_loop(0, 30, step, (lo, hi))
    thr = jax.lax.bitcast_convert_type(lo, jnp.float32)
    o_ref[...] = jnp.where(e >= thr, shifted - jnp.log(s), -70.0)


def _tc_part(x):
    B, V, C = x.shape
    xt = x.transpose(1, 0, 2).reshape(V, B * C)
    out = pl.pallas_call(
        _tc_body,
        grid=(B * C // 128,),
        in_specs=[pl.BlockSpec((V, 128), lambda c: (0, c))],
        out_specs=pl.BlockSpec((V, 128), lambda c: (0, c)),
        out_shape=jax.ShapeDtypeStruct((V, B * C), jnp.float32),
    )(xt)
    return out.reshape(V, B, C).transpose(1, 0, 2)


_B_SC = 32


def kernel(logits):
    out_sc = _sc_part(logits[:_B_SC])
    out_tc = _tc_part(logits[_B_SC:])
    return jnp.concatenate([out_sc, out_tc], axis=0)


# speedup vs baseline: 2.5674x; 1.0906x over previous
import functools

import jax
import jax.numpy as jnp
from jax import lax
from jax.experimental import pallas as pl
from jax.experimental.pallas import tpu as pltpu
from jax.experimental.pallas import tpu_sc as plsc

_R = 0.86
_L = 16
_NW = 32
_CH = 512
_NB = 2048
_HPAD = 2052
_NCAND = 520
_LN2 = 0.6931471805599453


def _ilog_poly(s):
    bits = plsc.bitcast(s, jnp.int32)
    ex = lax.shift_right_logical(bits, 23) - 127
    mb = lax.bitwise_or(lax.bitwise_and(bits, 0x7FFFFF), 0x3F800000)
    m = plsc.bitcast(mb, jnp.float32)
    z = (m - 1.0) / (m + 1.0)
    z2 = z * z
    p = 1.0 / 9.0 + z2 * 0.0
    p = 1.0 / 7.0 + z2 * p
    p = 1.0 / 5.0 + z2 * p
    p = 1.0 / 3.0 + z2 * p
    p = 1.0 + z2 * p
    return ex.astype(jnp.float32) * _LN2 + 2.0 * z * p


def _search(hist, col, a0, target, nbins):
    lo = jnp.zeros((_L,), jnp.int32)
    hi = jnp.full((_L,), nbins - 1, jnp.int32)
    steps = nbins.bit_length() - 1

    def step(_, carry):
        lo, hi = carry
        mid = lax.shift_right_logical(lo + hi + 1, 1)
        sv = plsc.load_gather(hist, [mid, col])
        c = (a0 + sv) >= target
        return jnp.where(c, mid, lo), jnp.where(c, hi, mid - 1)

    lo, hi = lax.fori_loop(0, steps, step, (lo, hi))
    anext = a0 + plsc.load_gather(hist, [lo + 1, col])
    return lo, anext


def _zero_hist2(hist, nrows):
    @plsc.parallel_loop(0, nrows, unroll=8)
    def z(j):
        hist[j, pl.ds(0, _L)] = jnp.zeros((_L,), jnp.float32)
        hist[j, pl.ds(_L, _L)] = jnp.zeros((_L,), jnp.float32)


def _zero_hist1(hist, g, nrows):
    @plsc.parallel_loop(0, nrows, unroll=8)
    def z(j):
        hist[j, pl.ds(g * _L, _L)] = jnp.zeros((_L,), jnp.float32)


def _suffix_sum2(hist, nbins):
    def sfx(j, accs):
        a0, a1 = accs
        jr = nbins - 1 - j
        a0 = a0 + hist[jr, pl.ds(0, _L)]
        a1 = a1 + hist[jr, pl.ds(_L, _L)]
        hist[jr, pl.ds(0, _L)] = a0
        hist[jr, pl.ds(_L, _L)] = a1
        return a0, a1

    zero = jnp.zeros((_L,), jnp.float32)
    return lax.fori_loop(0, nbins, sfx, (zero, zero), unroll=4)


def _suffix_sum1(hist, g, nbins):
    def sfx(j, acc):
        jr = nbins - 1 - j
        acc = acc + hist[jr, pl.ds(g * _L, _L)]
        hist[jr, pl.ds(g * _L, _L)] = acc
        return acc

    return lax.fori_loop(0, nbins, sfx, jnp.zeros((_L,), jnp.float32),
                         unroll=4)


def _sc_body(x_hbm, o_hbm, buf, hist, cand_e, cand_k):
    wid = lax.axis_index("s") * 2 + lax.axis_index("c")
    lane = lax.iota(jnp.int32, _L)
    B, V, C = x_hbm.shape
    nchunks = V // _CH

    def run_unit(t, _):
        bidx = t * _NW + wid

        _zero_hist2(hist, _HPAD)

        def scan1_chunk(ci, _):
            pltpu.sync_copy(x_hbm.at[bidx, pl.ds(ci * _CH, _CH)], buf)

            @plsc.parallel_loop(0, _CH, unroll=8)
            def row(i):
                e0 = jnp.exp(buf[i, pl.ds(0, _L)])
                e1 = jnp.exp(buf[i, pl.ds(_L, _L)])
                k0 = plsc.bitcast(e0, jnp.int32)
                k1 = plsc.bitcast(e1, jnp.int32)
                plsc.addupdate_scatter(
                    hist, [lax.shift_right_logical(k0, 20), lane], e0)
                plsc.addupdate_scatter(
                    hist, [lax.shift_right_logical(k1, 20), lane + _L], e1)

            return 0

        lax.fori_loop(0, nchunks, scan1_chunk, 0)

        s0, s1 = _suffix_sum2(hist, _NB)
        tg0, tg1 = _R * s0, _R * s1
        zero = jnp.zeros((_L,), jnp.float32)
        b1s0, a10 = _search(hist, lane, zero, tg0, _NB)
        b1s1, a11 = _search(hist, lane + _L, zero, tg1, _NB)

        def scan2_chunk(ci, cnts):
            pltpu.sync_copy(x_hbm.at[bidx, pl.ds(ci * _CH, _CH)], buf)

            @plsc.parallel_loop(0, _CH, unroll=8, carry=cnts)
            def row(i, cnts):
                c0, c1 = cnts
                e0 = jnp.exp(buf[i, pl.ds(0, _L)])
                e1 = jnp.exp(buf[i, pl.ds(_L, _L)])
                k0 = plsc.bitcast(e0, jnp.int32)
                k1 = plsc.bitcast(e1, jnp.int32)
                m0 = (lax.shift_right_logical(k0, 20) == b1s0) & (c0 < _NCAND)
                m1 = (lax.shift_right_logical(k1, 20) == b1s1) & (c1 < _NCAND)
                plsc.store_scatter(cand_e, [c0, lane], e0, mask=m0)
                plsc.store_scatter(cand_k, [c0, lane], k0, mask=m0)
                plsc.store_scatter(cand_e, [c1, lane + _L], e1, mask=m1)
                plsc.store_scatter(cand_k, [c1, lane + _L], k1, mask=m1)
                return (c0 + jnp.where(m0, 1, 0), c1 + jnp.where(m1, 1, 0))

            return row

        czero = jnp.zeros((_L,), jnp.int32)
        cnt0, cnt1 = lax.fori_loop(0, nchunks, scan2_chunk, (czero, czero))

        taus = []
        for g, (b1s, a1, cnt, tgt) in enumerate(
                ((b1s0, a10, cnt0, tg0), (b1s1, a11, cnt1, tg1))):
            col = lane + g * _L
            nmax = jnp.max(cnt)

            _zero_hist1(hist, g, 1028)

            def r2(j, _, cnt=cnt, col=col, g=g):
                valid = (lane * 0 + j) < cnt
                k = cand_k[j, pl.ds(g * _L, _L)]
                b2 = lax.bitwise_and(lax.shift_right_logical(k, 10), 0x3FF)
                plsc.addupdate_scatter(
                    hist, [b2, col], cand_e[j, pl.ds(g * _L, _L)], mask=valid)
                return 0

            lax.fori_loop(0, nmax, r2, 0)
            _suffix_sum1(hist, g, 1024)
            b2s, a2 = _search(hist, col, a1, tgt, 1024)

            _zero_hist1(hist, g, 1028)

            def r3(j, _, cnt=cnt, col=col, g=g, b2s=b2s):
                k = cand_k[j, pl.ds(g * _L, _L)]
                valid = ((lane * 0 + j) < cnt) & (
                    lax.bitwise_and(lax.shift_right_logical(k, 10), 0x3FF)
                    == b2s)
                b3 = lax.bitwise_and(k, 0x3FF)
                plsc.addupdate_scatter(
                    hist, [b3, col], cand_e[j, pl.ds(g * _L, _L)], mask=valid)
                return 0

            lax.fori_loop(0, nmax, r3, 0)
            _suffix_sum1(hist, g, 1024)
            b3s, _ = _search(hist, col, a2, tgt, 1024)

            tau_k = lax.bitwise_or(
                lax.bitwise_or(lax.shift_left(b1s, 20),
                               lax.shift_left(b2s, 10)), b3s)
            taus.append(plsc.bitcast(tau_k, jnp.float32))

        tau0, tau1 = taus
        logz0 = _ilog_poly(s0)
        logz1 = _ilog_poly(s1)

        def scan3_chunk(ci, _):
            pltpu.sync_copy(x_hbm.at[bidx, pl.ds(ci * _CH, _CH)], buf)

            @plsc.parallel_loop(0, _CH, unroll=8)
            def row(i):
                v0 = buf[i, pl.ds(0, _L)]
                v1 = buf[i, pl.ds(_L, _L)]
                e0 = jnp.exp(v0)
                e1 = jnp.exp(v1)
                cand_e[i, pl.ds(0, _L)] = jnp.where(
                    e0 >= tau0, v0 - logz0, -70.0)
                cand_e[i, pl.ds(_L, _L)] = jnp.where(
                    e1 >= tau1, v1 - logz1, -70.0)
            pltpu.sync_copy(cand_e.at[pl.ds(0, _CH)],
                            o_hbm.at[bidx, pl.ds(ci * _CH, _CH)])
            return 0

        lax.fori_loop(0, nchunks, scan3_chunk, 0)
        return 0

    lax.fori_loop(0, B // _NW, run_unit, 0)


def _sc_part(x):
    B, V, C = x.shape
    mesh = plsc.VectorSubcoreMesh(
        core_axis_name="c", subcore_axis_name="s", num_cores=2, num_subcores=16)
    f = pl.kernel(
        functools.partial(_sc_body),
        out_type=jax.ShapeDtypeStruct((B, V, C), jnp.float32),
        mesh=mesh,
        compiler_params=pltpu.CompilerParams(
            use_tc_tiling_on_sc=False, needs_layout_passes=False),
        scratch_types=[
            pltpu.VMEM((_CH, C), jnp.float32),
            pltpu.VMEM((_HPAD, C), jnp.float32),
            pltpu.VMEM((_NCAND, C), jnp.float32),
            pltpu.VMEM((_NCAND, C), jnp.int32),
        ],
    )
    return f(x)


def _tc_body(x_ref, o_ref):
    x = x_ref[...]
    m = jnp.max(x, axis=0, keepdims=True)
    shifted = x - m
    e = jnp.exp(shifted)
    s = jnp.sum(e, axis=0, keepdims=True)
    target = _R * s

    lo = jnp.zeros(s.shape, jnp.int32)
    hi = jnp.full(s.shape, 0x3F800000, jnp.int32)

    def step(_, carry):
        lo, hi = carry
        mid = lo + ((hi - lo) >> 1)
        mid_f = jax.lax.bitcast_convert_type(mid, jnp.float32)
        g = jnp.sum(jnp.where(e > mid_f, e, 0.0), axis=0, keepdims=True)
        cond = g < target
        return jnp.where(cond, lo, mid + 1), jnp.where(cond, mid, hi)

    lo, hi = lax.fori_loop(0, 30, step, (lo, hi))
    thr = jax.lax.bitcast_convert_type(lo, jnp.float32)
    o_ref[...] = jnp.where(e >= thr, shifted - jnp.log(s), -70.0)


def _tc_part(x):
    B, V, C = x.shape
    xt = x.transpose(1, 0, 2).reshape(V, B * C)
    out = pl.pallas_call(
        _tc_body,
        grid=(B * C // 128,),
        in_specs=[pl.BlockSpec((V, 128), lambda c: (0, c))],
        out_specs=pl.BlockSpec((V, 128), lambda c: (0, c)),
        out_shape=jax.ShapeDtypeStruct((V, B * C), jnp.float32),
    )(xt)
    return out.reshape(V, B, C).transpose(1, 0, 2)


_B_SC = 32


def kernel(logits):
    return _sc_part(logits)
